# baseline (speedup 1.0000x reference)
import jax
import jax.numpy as jnp
from jax import lax
from jax.experimental import pallas as pl
from jax.experimental.pallas import tpu as pltpu

B, T, NM, L, NL = 32, 1000, 80, 200, 4


def _body(ml_ref, mp_ref, mt_ref, go_ref, gt_ref, vm_ref, mel_len_ref,
          seq_len_ref, a2_ref, out_lin, out_post, out_gate, out_guide, acc):
    b = pl.program_id(0)

    @pl.when(b == 0)
    def _init():
        for i in range(6):
            acc[i] = 0.0

    valid = vm_ref[0]
    ml = ml_ref[0]
    mp = mp_ref[0]
    mt = mt_ref[0]

    rs_lin = jnp.sum(jnp.abs(ml - mt), axis=1).reshape(1, T)
    rs_post = jnp.sum(jnp.abs(mp - mt), axis=1).reshape(1, T)
    acc[0] += jnp.sum(rs_lin * valid)
    acc[1] += jnp.sum(rs_post * valid)

    x = go_ref[0]
    z = gt_ref[0]
    bce = jnp.maximum(x, 0.0) - x * z + jnp.log(1.0 + jnp.exp(-jnp.abs(x)))
    acc[2] += jnp.sum(bce * valid)
    acc[3] += jnp.sum(valid)

    t_i = mel_len_ref[b].astype(jnp.float32)
    l_i = seq_len_ref[b].astype(jnp.float32)
    inv_t = 1.0 / jnp.maximum(t_i, 1.0)
    inv_l = 1.0 / jnp.maximum(l_i, 1.0)
    t1 = lax.broadcasted_iota(jnp.int32, (T, L), 0).astype(jnp.float32) + 1.0
    l1 = lax.broadcasted_iota(jnp.int32, (T, L), 1).astype(jnp.float32) + 1.0
    d = a2_ref[0]
    dsum = d[0] + d[1]
    diff = t1 * inv_t - l1 * inv_l
    w = 1.0 - jnp.exp(-12.5 * (diff * diff))
    maskf = jnp.where((t1 <= t_i) & (l1 <= l_i), 1.0, 0.0)
    acc[4] += jnp.sum(dsum * w * maskf)
    acc[5] += jnp.sum(maskf)

    @pl.when(b == B - 1)
    def _fin():
        vcount = jnp.maximum(acc[3], 1.0)
        out_lin[0, 0] = acc[0] / (vcount * NM)
        out_post[0, 0] = acc[1] / (vcount * NM)
        out_gate[0, 0] = acc[2] / vcount
        den = jnp.maximum(2.0 * acc[5], 1.0)
        out_guide[0, 0] = 10.0 * acc[4] / den


def kernel(mel_linear, mel_post, gate_out, mel_target, gate_target, mel_mask,
           mel_len, seq_len, alignments2):
    valid = 1.0 - mel_mask.astype(jnp.float32)
    scalar_shape = jax.ShapeDtypeStruct((1, 1), jnp.float32)
    smem_scalar = pl.BlockSpec((1, 1), lambda b: (0, 0), memory_space=pltpu.SMEM)
    outs = pl.pallas_call(
        _body,
        grid=(B,),
        in_specs=[
            pl.BlockSpec((1, T, NM), lambda b: (b, 0, 0)),
            pl.BlockSpec((1, T, NM), lambda b: (b, 0, 0)),
            pl.BlockSpec((1, T, NM), lambda b: (b, 0, 0)),
            pl.BlockSpec((1, 1, T), lambda b: (b, 0, 0)),
            pl.BlockSpec((1, 1, T), lambda b: (b, 0, 0)),
            pl.BlockSpec((1, 1, T), lambda b: (b, 0, 0)),
            pl.BlockSpec(memory_space=pltpu.SMEM),
            pl.BlockSpec(memory_space=pltpu.SMEM),
            pl.BlockSpec((1, 2, T, L), lambda b: (b, 1, 0, 0)),
        ],
        out_specs=[smem_scalar] * 4,
        out_shape=[scalar_shape] * 4,
        scratch_shapes=[pltpu.SMEM((6,), jnp.float32)],
    )(mel_linear, mel_post, mel_target,
      gate_out.reshape(B, 1, T), gate_target.reshape(B, 1, T),
      valid.reshape(B, 1, T),
      mel_len.astype(jnp.int32), seq_len.astype(jnp.int32), alignments2)
    return tuple(o[0, 0] for o in outs)
